# baseline (speedup 1.0000x reference)
import functools

import jax
import jax.numpy as jnp
from jax.experimental import pallas as pl

BM = 512


def _matmul_kernel(x_ref, w0_ref, w1_ref, o_ref):
    j = pl.program_id(1)

    @pl.when(j == 0)
    def _():
        o_ref[...] = jax.lax.dot_general(
            x_ref[...], w0_ref[...], (((0,), (1,)), ((), ())),
            preferred_element_type=jnp.float32,
        )

    @pl.when(j == 1)
    def _():
        o_ref[...] = jax.lax.dot_general(
            x_ref[...], w1_ref[...], (((0,), (1,)), ((), ())),
            preferred_element_type=jnp.float32,
        )


@functools.partial(jax.jit, static_argnames=())
def kernel(input, weight):
    m, k = weight.shape
    _, n = input.shape
    half = m // 2 // BM
    out_t = pl.pallas_call(
        _matmul_kernel,
        grid=(half, 2),
        in_specs=[
            pl.BlockSpec((k, n), lambda i, j: (0, 0)),
            pl.BlockSpec((BM, k), lambda i, j: (i, 0)),
            pl.BlockSpec((BM, k), lambda i, j: (half + i, 0)),
        ],
        out_specs=pl.BlockSpec((n, BM), lambda i, j: (0, i + j * half)),
        out_shape=jax.ShapeDtypeStruct((n, m), jnp.float32),
    )(input, weight, weight)
    return out_t.T


# speedup vs baseline: 1.2331x; 1.2331x over previous
import functools

import jax
import jax.numpy as jnp
from jax.experimental import pallas as pl
from jax.experimental.pallas import tpu as pltpu

BM = 512
NBUF = 6


def _body(x_ref, w_ref, o_ref, *scratch):
    bufs = scratch[:NBUF]
    sems = scratch[NBUF:]
    m = w_ref.shape[0]
    nchunks = m // BM

    def start(i):
        pltpu.make_async_copy(
            w_ref.at[pl.ds(i * BM, BM), :], bufs[i % NBUF], sems[i % NBUF]
        ).start()

    for i in range(min(NBUF, nchunks)):
        start(i)
    for i in range(nchunks):
        pltpu.make_async_copy(
            w_ref.at[pl.ds(i * BM, BM), :], bufs[i % NBUF], sems[i % NBUF]
        ).wait()
        o_ref[:, pl.ds(i * BM, BM)] = jax.lax.dot_general(
            x_ref[...], bufs[i % NBUF][...], (((0,), (1,)), ((), ())),
            preferred_element_type=jnp.float32,
        )
        if i + NBUF < nchunks:
            start(i + NBUF)


@functools.partial(jax.jit, static_argnames=())
def kernel(input, weight):
    m, k = weight.shape
    _, n = input.shape
    out_t = pl.pallas_call(
        _body,
        in_specs=[
            pl.BlockSpec(memory_space=pltpu.MemorySpace.VMEM),
            pl.BlockSpec(memory_space=pltpu.MemorySpace.HBM),
        ],
        out_specs=pl.BlockSpec(memory_space=pltpu.MemorySpace.VMEM),
        out_shape=jax.ShapeDtypeStruct((n, m), jnp.float32),
        scratch_shapes=(
            [pltpu.VMEM((BM, k), jnp.float32) for _ in range(NBUF)]
            + [pltpu.SemaphoreType.DMA for _ in range(NBUF)]
        ),
    )(input, weight)
    return out_t.T
